# initial kernel scaffold (unmeasured)
import jax
import jax.numpy as jnp
from jax import lax
from jax.experimental import pallas as pl
from jax.experimental.pallas import tpu as pltpu


def kernel(
    x,
):
    def body(*refs):
        pass

    out_shape = jax.ShapeDtypeStruct(..., jnp.float32)
    return pl.pallas_call(body, out_shape=out_shape)(...)


# baseline (device time: 47169 ns/iter reference)
import jax
import jax.numpy as jnp
from jax import lax
from jax.experimental import pallas as pl
from jax.experimental.pallas import tpu as pltpu

N_DEV = 4


def kernel(x):
    _, m, n_total = x.shape
    n_out = n_total // N_DEV

    def body(x_ref, out_ref, comm_ref, send_sems, recv_sems):
        my = lax.axis_index("i")
        left = (my + N_DEV - 1) % N_DEV
        right = (my + 1) % N_DEV

        barrier_sem = pltpu.get_barrier_semaphore()
        for nbr in [left, right]:
            pl.semaphore_signal(
                barrier_sem, inc=1,
                device_id=(nbr,), device_id_type=pl.DeviceIdType.MESH,
            )
        pl.semaphore_wait(barrier_sem, 2)

        def local_chunk_f32(c):
            return x_ref[0, :, pl.ds(c * n_out, n_out)]

        comm_ref[0, :, :] = local_chunk_f32(
            (my + N_DEV - 1) % N_DEV
        ).astype(jnp.bfloat16)

        for h in range(N_DEV - 1):
            send_slot = h % 2
            recv_slot = (h + 1) % 2
            rdma = pltpu.make_async_remote_copy(
                src_ref=comm_ref.at[send_slot],
                dst_ref=comm_ref.at[recv_slot],
                send_sem=send_sems.at[send_slot],
                recv_sem=recv_sems.at[recv_slot],
                device_id=(right,),
                device_id_type=pl.DeviceIdType.MESH,
            )
            rdma.start()
            rdma.wait()

            c = (my + 2 * N_DEV - 2 - h) % N_DEV
            if h < N_DEV - 2:
                comm_ref[recv_slot, :, :] = (
                    comm_ref[recv_slot, :, :]
                    + local_chunk_f32(c).astype(jnp.bfloat16)
                )
            else:
                out_ref[:, :] = (
                    comm_ref[recv_slot, :, :].astype(jnp.float32)
                    + local_chunk_f32(c)
                )

    return pl.pallas_call(
        body,
        out_shape=jax.ShapeDtypeStruct((m, n_out), jnp.float32),
        in_specs=[pl.BlockSpec(memory_space=pltpu.VMEM)],
        out_specs=pl.BlockSpec(memory_space=pltpu.VMEM),
        scratch_shapes=[
            pltpu.VMEM((2, m, n_out), jnp.bfloat16),
            pltpu.SemaphoreType.DMA((2,)),
            pltpu.SemaphoreType.DMA((2,)),
        ],
        compiler_params=pltpu.CompilerParams(collective_id=0),
    )(x)


# device time: 30549 ns/iter; 1.5440x vs baseline; 1.5440x over previous
import jax
import jax.numpy as jnp
from jax import lax
from jax.experimental import pallas as pl
from jax.experimental.pallas import tpu as pltpu

N_DEV = 4


def kernel(x):
    _, m, n_total = x.shape
    n_out = n_total // N_DEV
    n_half = n_out // 2

    def body(
        x_ref, out_ref,
        comm_r, comm_l,
        send_sems_r, recv_sems_r, send_sems_l, recv_sems_l,
    ):
        my = lax.axis_index("i")
        left = (my + N_DEV - 1) % N_DEV
        right = (my + 1) % N_DEV

        barrier_sem = pltpu.get_barrier_semaphore()
        for nbr in [left, right]:
            pl.semaphore_signal(
                barrier_sem, inc=1,
                device_id=(nbr,), device_id_type=pl.DeviceIdType.MESH,
            )
        pl.semaphore_wait(barrier_sem, 2)

        def lhalf_f32(c):
            return x_ref[0, :, pl.ds(c * n_out, n_half)]

        def rhalf_f32(c):
            return x_ref[0, :, pl.ds(c * n_out + n_half, n_half)]

        comm_r[0, :, :] = lhalf_f32((my + N_DEV - 1) % N_DEV).astype(jnp.bfloat16)
        comm_l[0, :, :] = rhalf_f32((my + 1) % N_DEV).astype(jnp.bfloat16)

        for h in range(N_DEV - 1):
            s_slot = h % 2
            r_slot = (h + 1) % 2
            rdma_r = pltpu.make_async_remote_copy(
                src_ref=comm_r.at[s_slot],
                dst_ref=comm_r.at[r_slot],
                send_sem=send_sems_r.at[s_slot],
                recv_sem=recv_sems_r.at[r_slot],
                device_id=(right,),
                device_id_type=pl.DeviceIdType.MESH,
            )
            rdma_l = pltpu.make_async_remote_copy(
                src_ref=comm_l.at[s_slot],
                dst_ref=comm_l.at[r_slot],
                send_sem=send_sems_l.at[s_slot],
                recv_sem=recv_sems_l.at[r_slot],
                device_id=(left,),
                device_id_type=pl.DeviceIdType.MESH,
            )
            rdma_r.start()
            rdma_l.start()
            rdma_r.wait()
            rdma_l.wait()

            c_r = (my + 2 * N_DEV - 2 - h) % N_DEV
            c_l = (my + 2 + h) % N_DEV
            if h < N_DEV - 2:
                comm_r[r_slot, :, :] = (
                    comm_r[r_slot, :, :] + lhalf_f32(c_r).astype(jnp.bfloat16)
                )
                comm_l[r_slot, :, :] = (
                    comm_l[r_slot, :, :] + rhalf_f32(c_l).astype(jnp.bfloat16)
                )
            else:
                out_ref[:, pl.ds(0, n_half)] = (
                    comm_r[r_slot, :, :].astype(jnp.float32) + lhalf_f32(my)
                )
                out_ref[:, pl.ds(n_half, n_half)] = (
                    comm_l[r_slot, :, :].astype(jnp.float32) + rhalf_f32(my)
                )

    return pl.pallas_call(
        body,
        out_shape=jax.ShapeDtypeStruct((m, n_out), jnp.float32),
        in_specs=[pl.BlockSpec(memory_space=pltpu.VMEM)],
        out_specs=pl.BlockSpec(memory_space=pltpu.VMEM),
        scratch_shapes=[
            pltpu.VMEM((2, m, n_half), jnp.bfloat16),
            pltpu.VMEM((2, m, n_half), jnp.bfloat16),
            pltpu.SemaphoreType.DMA((2,)),
            pltpu.SemaphoreType.DMA((2,)),
            pltpu.SemaphoreType.DMA((2,)),
            pltpu.SemaphoreType.DMA((2,)),
        ],
        compiler_params=pltpu.CompilerParams(collective_id=0),
    )(x)


# device time: 26780 ns/iter; 1.7614x vs baseline; 1.1407x over previous
import jax
import jax.numpy as jnp
from jax import lax
from jax.experimental import pallas as pl
from jax.experimental.pallas import tpu as pltpu

N_DEV = 4
N_HOP = N_DEV - 1
N_SUB = 2


def kernel(x):
    _, m, n_total = x.shape
    n_out = n_total // N_DEV
    n_half = n_out // 2
    m_sub = m // N_SUB

    def body(
        x_ref, out_ref,
        comm_r, comm_l,
        send_sems_r, recv_sems_r, send_sems_l, recv_sems_l,
    ):
        my = lax.axis_index("i")
        left = (my + N_DEV - 1) % N_DEV
        right = (my + 1) % N_DEV

        barrier_sem = pltpu.get_barrier_semaphore()
        for nbr in [left, right]:
            pl.semaphore_signal(
                barrier_sem, inc=1,
                device_id=(nbr,), device_id_type=pl.DeviceIdType.MESH,
            )
        pl.semaphore_wait(barrier_sem, 2)

        def lhalf_f32(c, j):
            return x_ref[0, pl.ds(j * m_sub, m_sub), pl.ds(c * n_out, n_half)]

        def rhalf_f32(c, j):
            return x_ref[
                0, pl.ds(j * m_sub, m_sub), pl.ds(c * n_out + n_half, n_half)
            ]

        def make_rdma(comm, send_sems, recv_sems, h, j, dst):
            return pltpu.make_async_remote_copy(
                src_ref=comm.at[h, pl.ds(j * m_sub, m_sub), :],
                dst_ref=comm.at[h + 1, pl.ds(j * m_sub, m_sub), :],
                send_sem=send_sems.at[h, j],
                recv_sem=recv_sems.at[h + 1, j],
                device_id=(dst,),
                device_id_type=pl.DeviceIdType.MESH,
            )

        c0_r = (my + N_DEV - 1) % N_DEV
        c0_l = (my + 1) % N_DEV
        sends = []
        for j in range(N_SUB):
            comm_r[0, pl.ds(j * m_sub, m_sub), :] = lhalf_f32(c0_r, j).astype(
                jnp.bfloat16
            )
            rd = make_rdma(comm_r, send_sems_r, recv_sems_r, 0, j, right)
            rd.start()
            sends.append(rd)
            comm_l[0, pl.ds(j * m_sub, m_sub), :] = rhalf_f32(c0_l, j).astype(
                jnp.bfloat16
            )
            ld = make_rdma(comm_l, send_sems_l, recv_sems_l, 0, j, left)
            ld.start()
            sends.append(ld)

        for h in range(1, N_HOP + 1):
            c_r = (my + 2 * N_DEV - 1 - h) % N_DEV
            c_l = (my + 1 + h) % N_DEV
            for j in range(N_SUB):
                rrecv = make_rdma(comm_r, send_sems_r, recv_sems_r, h - 1, j, right)
                rrecv.wait_recv()
                rows = pl.ds(j * m_sub, m_sub)
                if h < N_HOP:
                    comm_r[h, rows, :] = (
                        comm_r[h, rows, :] + lhalf_f32(c_r, j).astype(jnp.bfloat16)
                    )
                    rd = make_rdma(comm_r, send_sems_r, recv_sems_r, h, j, right)
                    rd.start()
                    sends.append(rd)
                else:
                    out_ref[rows, pl.ds(0, n_half)] = (
                        comm_r[h, rows, :].astype(jnp.float32) + lhalf_f32(my, j)
                    )

                lrecv = make_rdma(comm_l, send_sems_l, recv_sems_l, h - 1, j, left)
                lrecv.wait_recv()
                if h < N_HOP:
                    comm_l[h, rows, :] = (
                        comm_l[h, rows, :] + rhalf_f32(c_l, j).astype(jnp.bfloat16)
                    )
                    ld = make_rdma(comm_l, send_sems_l, recv_sems_l, h, j, left)
                    ld.start()
                    sends.append(ld)
                else:
                    out_ref[rows, pl.ds(n_half, n_half)] = (
                        comm_l[h, rows, :].astype(jnp.float32) + rhalf_f32(my, j)
                    )

        for rd in sends:
            rd.wait_send()

    return pl.pallas_call(
        body,
        out_shape=jax.ShapeDtypeStruct((m, n_out), jnp.float32),
        in_specs=[pl.BlockSpec(memory_space=pltpu.VMEM)],
        out_specs=pl.BlockSpec(memory_space=pltpu.VMEM),
        scratch_shapes=[
            pltpu.VMEM((N_HOP + 1, m, n_half), jnp.bfloat16),
            pltpu.VMEM((N_HOP + 1, m, n_half), jnp.bfloat16),
            pltpu.SemaphoreType.DMA((N_HOP + 1, N_SUB)),
            pltpu.SemaphoreType.DMA((N_HOP + 1, N_SUB)),
            pltpu.SemaphoreType.DMA((N_HOP + 1, N_SUB)),
            pltpu.SemaphoreType.DMA((N_HOP + 1, N_SUB)),
        ],
        compiler_params=pltpu.CompilerParams(collective_id=0),
    )(x)
